# Initial kernel scaffold; baseline (speedup 1.0000x reference)
import jax
import jax.numpy as jnp
from jax.experimental import pallas as pl


def kernel(hidden, labels, mask, W, b, start_trans, end_trans, trans):
    raise NotImplementedError("write your pallas kernel here")


# speedup vs baseline: 1.5253x; 1.5253x over previous
import functools

import jax
import jax.numpy as jnp
from jax.experimental import pallas as pl
from jax.experimental.pallas import tpu as pltpu

_NEG = -1e30


def _em_kernel(hid_ref, w_ref, b_ref, em_ref, emt_ref):
    x = hid_ref[...]
    x = jnp.where(x >= 0, x, x * 0.01)
    bb, sb, h = x.shape
    y = jnp.dot(x.reshape(bb * sb, h), w_ref[...],
                preferred_element_type=jnp.float32)
    y = y + b_ref[...]
    y3 = y.reshape(bb, sb, 128)
    em_ref[...] = y3[:, :, :em_ref.shape[2]]
    emt_ref[...] = y3.transpose(1, 0, 2)


def _crf_kernel(emt_ref, mbc_ref, lbc_ref, start_ref, end_ref, trans_ref,
                llh_ref, tags_ref,
                score_s, vscore_s, numvec_s, lt_s, hist_s, texp_s, transt_s,
                tmax_s):
    B = emt_ref.shape[1]
    S = emt_ref.shape[0]
    T128 = 128

    trans_m = trans_ref[...]
    tmax = jnp.max(trans_m, axis=0, keepdims=True)
    texp_s[...] = jnp.exp(trans_m - tmax)
    transt_s[...] = trans_m.T
    tmax_s[...] = jnp.broadcast_to(tmax, (8, T128))

    rowi = jax.lax.broadcasted_iota(jnp.int32, (B, T128), 1)
    em0 = emt_ref[0]
    lab0 = lbc_ref[0].astype(jnp.int32)
    startv = start_ref[...]
    score_s[...] = startv + em0
    vscore_s[...] = startv + em0
    numvec_s[...] = jnp.where(rowi == lab0, em0 + startv, 0.0)
    lt_s[...] = lab0

    def body(s, _):
        em_i = emt_ref[s]
        mb = mbc_ref[s].astype(jnp.int32) > 0
        lab = lbc_ref[s].astype(jnp.int32)
        labp = lbc_ref[s - 1].astype(jnp.int32)
        score = score_s[...]
        smax = jnp.max(score, axis=1, keepdims=True)
        p = jnp.exp(score - smax)
        m = jnp.dot(p, texp_s[...], preferred_element_type=jnp.float32)
        lse = smax + tmax_s[0:1, :] + jnp.log(m)
        score_s[...] = jnp.where(mb, lse + em_i, score)
        vscore = vscore_s[...]
        cand = vscore[:, None, :] + transt_s[...][None, :, :]
        vidx = jnp.argmax(cand, axis=2)
        vmx = jnp.max(cand, axis=2)
        vscore_s[...] = jnp.where(mb, vmx + em_i, vscore)
        vidx = jnp.where(mb, vidx, rowi)
        hist_s[pl.ds(s - 1, 1)] = vidx.reshape(1, B, T128)
        ohp = (rowi == labp).astype(jnp.float32)
        r = jnp.dot(ohp, trans_m, preferred_element_type=jnp.float32)
        numvec_s[...] += jnp.where((rowi == lab) & mb, em_i + r, 0.0)
        lt_s[...] = jnp.where(mb, lab, lt_s[...])
        return 0

    jax.lax.fori_loop(1, S, body, 0)

    endv = end_ref[...]
    x = score_s[...] + endv
    dmax = jnp.max(x, axis=1, keepdims=True)
    denom = dmax + jnp.log(jnp.sum(jnp.exp(x - dmax), axis=1, keepdims=True))
    numvec = numvec_s[...] + jnp.where(rowi == lt_s[...],
                                       jnp.broadcast_to(endv, (B, T128)), 0.0)
    num = jnp.sum(numvec, axis=1, keepdims=True)
    llh_ref[...] = num - denom
    vx = vscore_s[...] + endv
    best = jnp.argmax(vx, axis=1, keepdims=True)
    best_bc = jnp.broadcast_to(best, (B, T128))
    mlast = mbc_ref[S - 1].astype(jnp.int32) > 0
    tags_ref[pl.ds(S - 1, 1)] = jnp.where(mlast, best_bc, 0).astype(
        jnp.int8).reshape(1, B, T128)

    def back(r, carry):
        t = S - 2 - r
        h = hist_s[pl.ds(t, 1)].reshape(B, T128)
        prev = jnp.sum(jnp.where(rowi == carry, h, 0), axis=1, keepdims=True)
        prev_bc = jnp.broadcast_to(prev, (B, T128))
        mt = mbc_ref[t].astype(jnp.int32) > 0
        tags_ref[pl.ds(t, 1)] = jnp.where(mt, prev_bc, 0).astype(
            jnp.int8).reshape(1, B, T128)
        return prev_bc

    jax.lax.fori_loop(0, S - 1, back, best_bc)


def kernel(hidden, labels, mask, W, b, start_trans, end_trans, trans):
    B, S, H = hidden.shape
    T = W.shape[1]
    PT = 128 - T

    w_pad = jnp.pad(W, ((0, 0), (0, PT)))
    b_pad = jnp.pad(b, (0, PT), constant_values=_NEG).reshape(1, 128)

    sb = 32
    em, emt = pl.pallas_call(
        _em_kernel,
        out_shape=(
            jax.ShapeDtypeStruct((B, S, T), jnp.float32),
            jax.ShapeDtypeStruct((S, B, 128), jnp.float32),
        ),
        grid=(S // sb,),
        in_specs=[
            pl.BlockSpec((B, sb, H), lambda j: (0, j, 0)),
            pl.BlockSpec((H, 128), lambda j: (0, 0)),
            pl.BlockSpec((1, 128), lambda j: (0, 0)),
        ],
        out_specs=(
            pl.BlockSpec((B, sb, T), lambda j: (0, j, 0)),
            pl.BlockSpec((sb, B, 128), lambda j: (j, 0, 0)),
        ),
        compiler_params=pltpu.CompilerParams(
            dimension_semantics=("arbitrary",),
        ),
        name="em_matmul",
    )(hidden, w_pad, b_pad)

    start_pad = jnp.pad(start_trans, (0, PT), constant_values=_NEG).reshape(1, 128)
    end_pad = jnp.pad(end_trans, (0, PT), constant_values=_NEG).reshape(1, 128)
    trans_pad = jnp.pad(trans, ((0, PT), (0, PT)), constant_values=_NEG)
    mbc = jnp.broadcast_to(
        mask.astype(jnp.int8).T[:, :, None], (S, B, 128))
    lbc = jnp.broadcast_to(
        labels.astype(jnp.int8).T[:, :, None], (S, B, 128))

    llh, tags_raw = pl.pallas_call(
        _crf_kernel,
        out_shape=(
            jax.ShapeDtypeStruct((B, 1), jnp.float32),
            jax.ShapeDtypeStruct((S, B, 128), jnp.int8),
        ),
        grid=(1,),
        in_specs=[
            pl.BlockSpec((S, B, 128), lambda i: (0, 0, 0)),
            pl.BlockSpec((S, B, 128), lambda i: (0, 0, 0)),
            pl.BlockSpec((S, B, 128), lambda i: (0, 0, 0)),
            pl.BlockSpec((1, 128), lambda i: (0, 0)),
            pl.BlockSpec((1, 128), lambda i: (0, 0)),
            pl.BlockSpec((128, 128), lambda i: (0, 0)),
        ],
        out_specs=(
            pl.BlockSpec((B, 1), lambda i: (0, 0)),
            pl.BlockSpec((S, B, 128), lambda i: (0, 0, 0)),
        ),
        scratch_shapes=[
            pltpu.VMEM((B, 128), jnp.float32),
            pltpu.VMEM((B, 128), jnp.float32),
            pltpu.VMEM((B, 128), jnp.float32),
            pltpu.VMEM((B, 128), jnp.int32),
            pltpu.VMEM((S - 1, B, 128), jnp.int32),
            pltpu.VMEM((128, 128), jnp.float32),
            pltpu.VMEM((128, 128), jnp.float32),
            pltpu.VMEM((8, 128), jnp.float32),
        ],
        compiler_params=pltpu.CompilerParams(
            dimension_semantics=("arbitrary",),
            vmem_limit_bytes=56 * 1024 * 1024,
        ),
        name="crf_fused",
    )(emt, mbc, lbc, start_pad, end_pad, trans_pad)

    loss = -jnp.sum(llh) / B
    tags = tags_raw[:, :, 0].astype(jnp.int32).T
    return loss, tags, em


# speedup vs baseline: 4.4345x; 2.9074x over previous
import functools

import jax
import jax.numpy as jnp
from jax.experimental import pallas as pl
from jax.experimental.pallas import tpu as pltpu

_NEG = -1e30


def _em_kernel(hid_ref, w_ref, b_ref, em_ref, emt_ref):
    x = hid_ref[...]
    x = jnp.where(x >= 0, x, x * 0.01)
    bb, sb, h = x.shape
    y = jnp.dot(x.reshape(bb * sb, h), w_ref[...],
                preferred_element_type=jnp.float32)
    y = y + b_ref[...]
    y3 = y.reshape(bb, sb, 128)
    em_ref[...] = y3[:, :, :em_ref.shape[2]]
    emt_ref[...] = y3.transpose(1, 0, 2)


def _crf_kernel(emt_ref, mbc_ref, lbc_ref, start_ref, end_ref, trans_ref,
                llh_ref, tags_ref,
                score_s, vscore_s, numvec_s, lt_s, hist_s, texp_s, transt_s,
                tmax_s):
    B = emt_ref.shape[1]
    S = emt_ref.shape[0]
    T128 = 128

    trans_m = trans_ref[...]
    tmax = jnp.max(trans_m, axis=0, keepdims=True)
    texp_s[...] = jnp.exp(trans_m - tmax)
    transt_s[...] = trans_m.T
    tmax_s[...] = jnp.broadcast_to(tmax, (8, T128))

    rowi = jax.lax.broadcasted_iota(jnp.int32, (B, T128), 1)
    em0 = emt_ref[0]
    lab0 = lbc_ref[0].astype(jnp.int32)
    startv = start_ref[...]
    score_s[...] = startv + em0
    vscore_s[...] = startv + em0
    numvec_s[...] = jnp.where(rowi == lab0, em0 + startv, 0.0)
    lt_s[...] = lab0

    def body(s, _):
        em_i = emt_ref[s]
        mb = mbc_ref[s].astype(jnp.int32) > 0
        lab = lbc_ref[s].astype(jnp.int32)
        labp = lbc_ref[s - 1].astype(jnp.int32)
        score = score_s[...]
        smax = jnp.max(score, axis=1, keepdims=True)
        p = jnp.exp(score - smax)
        m = jnp.dot(p, texp_s[...], preferred_element_type=jnp.float32)
        lse = smax + tmax_s[0:1, :] + jnp.log(m)
        score_s[...] = jnp.where(mb, lse + em_i, score)
        vscore = vscore_s[...]
        vmx_parts = []
        vidx_parts = []
        for jt in range(9):
            tt = transt_s[jt * 8:(jt + 1) * 8, :]
            cand_t = vscore[:, None, :] + tt[None, :, :]
            vmx_parts.append(jnp.max(cand_t, axis=2))
            vidx_parts.append(jnp.argmax(cand_t, axis=2))
        vmx = jnp.concatenate(vmx_parts, axis=1)
        vidx = jnp.concatenate(vidx_parts, axis=1)
        vmx = jnp.pad(vmx, ((0, 0), (0, T128 - 72)),
                      constant_values=_NEG)
        vidx = jnp.pad(vidx, ((0, 0), (0, T128 - 72)))
        vscore_s[...] = jnp.where(mb, vmx + em_i, vscore)
        vidx = jnp.where(mb, vidx, rowi)
        hist_s[pl.ds(s - 1, 1)] = vidx.reshape(1, B, T128)
        ohp = (rowi == labp).astype(jnp.float32)
        r = jnp.dot(ohp, trans_m, preferred_element_type=jnp.float32)
        numvec_s[...] += jnp.where((rowi == lab) & mb, em_i + r, 0.0)
        lt_s[...] = jnp.where(mb, lab, lt_s[...])
        return 0

    jax.lax.fori_loop(1, S, body, 0)

    endv = end_ref[...]
    x = score_s[...] + endv
    dmax = jnp.max(x, axis=1, keepdims=True)
    denom = dmax + jnp.log(jnp.sum(jnp.exp(x - dmax), axis=1, keepdims=True))
    numvec = numvec_s[...] + jnp.where(rowi == lt_s[...],
                                       jnp.broadcast_to(endv, (B, T128)), 0.0)
    num = jnp.sum(numvec, axis=1, keepdims=True)
    llh_ref[...] = num - denom
    vx = vscore_s[...] + endv
    best = jnp.argmax(vx, axis=1, keepdims=True)
    best_bc = jnp.broadcast_to(best, (B, T128))
    mlast = mbc_ref[S - 1].astype(jnp.int32) > 0
    tags_ref[pl.ds(S - 1, 1)] = jnp.where(mlast, best_bc, 0).astype(
        jnp.int8).reshape(1, B, T128)

    def back(r, carry):
        t = S - 2 - r
        h = hist_s[pl.ds(t, 1)].reshape(B, T128)
        prev = jnp.sum(jnp.where(rowi == carry, h, 0), axis=1, keepdims=True)
        prev_bc = jnp.broadcast_to(prev, (B, T128))
        mt = mbc_ref[t].astype(jnp.int32) > 0
        tags_ref[pl.ds(t, 1)] = jnp.where(mt, prev_bc, 0).astype(
            jnp.int8).reshape(1, B, T128)
        return prev_bc

    jax.lax.fori_loop(0, S - 1, back, best_bc)


def kernel(hidden, labels, mask, W, b, start_trans, end_trans, trans):
    B, S, H = hidden.shape
    T = W.shape[1]
    PT = 128 - T

    w_pad = jnp.pad(W, ((0, 0), (0, PT)))
    b_pad = jnp.pad(b, (0, PT), constant_values=_NEG).reshape(1, 128)

    sb = 32
    em, emt = pl.pallas_call(
        _em_kernel,
        out_shape=(
            jax.ShapeDtypeStruct((B, S, T), jnp.float32),
            jax.ShapeDtypeStruct((S, B, 128), jnp.float32),
        ),
        grid=(S // sb,),
        in_specs=[
            pl.BlockSpec((B, sb, H), lambda j: (0, j, 0)),
            pl.BlockSpec((H, 128), lambda j: (0, 0)),
            pl.BlockSpec((1, 128), lambda j: (0, 0)),
        ],
        out_specs=(
            pl.BlockSpec((B, sb, T), lambda j: (0, j, 0)),
            pl.BlockSpec((sb, B, 128), lambda j: (j, 0, 0)),
        ),
        compiler_params=pltpu.CompilerParams(
            dimension_semantics=("arbitrary",),
        ),
        name="em_matmul",
    )(hidden, w_pad, b_pad)

    start_pad = jnp.pad(start_trans, (0, PT), constant_values=_NEG).reshape(1, 128)
    end_pad = jnp.pad(end_trans, (0, PT), constant_values=_NEG).reshape(1, 128)
    trans_pad = jnp.pad(trans, ((0, PT), (0, PT)), constant_values=_NEG)
    mbc = jnp.broadcast_to(
        mask.astype(jnp.int8).T[:, :, None], (S, B, 128))
    lbc = jnp.broadcast_to(
        labels.astype(jnp.int8).T[:, :, None], (S, B, 128))

    llh, tags_raw = pl.pallas_call(
        _crf_kernel,
        out_shape=(
            jax.ShapeDtypeStruct((B, 1), jnp.float32),
            jax.ShapeDtypeStruct((S, B, 128), jnp.int8),
        ),
        grid=(1,),
        in_specs=[
            pl.BlockSpec((S, B, 128), lambda i: (0, 0, 0)),
            pl.BlockSpec((S, B, 128), lambda i: (0, 0, 0)),
            pl.BlockSpec((S, B, 128), lambda i: (0, 0, 0)),
            pl.BlockSpec((1, 128), lambda i: (0, 0)),
            pl.BlockSpec((1, 128), lambda i: (0, 0)),
            pl.BlockSpec((128, 128), lambda i: (0, 0)),
        ],
        out_specs=(
            pl.BlockSpec((B, 1), lambda i: (0, 0)),
            pl.BlockSpec((S, B, 128), lambda i: (0, 0, 0)),
        ),
        scratch_shapes=[
            pltpu.VMEM((B, 128), jnp.float32),
            pltpu.VMEM((B, 128), jnp.float32),
            pltpu.VMEM((B, 128), jnp.float32),
            pltpu.VMEM((B, 128), jnp.int32),
            pltpu.VMEM((S - 1, B, 128), jnp.int32),
            pltpu.VMEM((128, 128), jnp.float32),
            pltpu.VMEM((128, 128), jnp.float32),
            pltpu.VMEM((8, 128), jnp.float32),
        ],
        compiler_params=pltpu.CompilerParams(
            dimension_semantics=("arbitrary",),
            vmem_limit_bytes=56 * 1024 * 1024,
        ),
        name="crf_fused",
    )(emt, mbc, lbc, start_pad, end_pad, trans_pad)

    loss = -jnp.sum(llh) / B
    tags = tags_raw[:, :, 0].astype(jnp.int32).T
    return loss, tags, em


# speedup vs baseline: 6.0400x; 1.3620x over previous
import functools

import jax
import jax.numpy as jnp
from jax.experimental import pallas as pl
from jax.experimental.pallas import tpu as pltpu

_NEG = -1e30


def _em_kernel(hid_ref, w_ref, b_ref, em_ref, emt_ref):
    x = hid_ref[...]
    x = jnp.where(x >= 0, x, x * 0.01)
    bb, sb, h = x.shape
    y = jnp.dot(x.reshape(bb * sb, h), w_ref[...],
                preferred_element_type=jnp.float32)
    y = y + b_ref[...]
    y3 = y.reshape(bb, sb, 128)
    em_ref[...] = y3[:, :, :em_ref.shape[2]]
    emt_ref[...] = y3.transpose(1, 0, 2)


def _crf_kernel(emt_ref, mbc_ref, lbc_ref, start_ref, end_ref, trans_ref,
                llh_ref, tags_ref,
                score_s, vscore_s, numvec_s, lt_s, hist_s, texp_s, transt_s,
                tmax_s):
    B = emt_ref.shape[1]
    S = emt_ref.shape[0]
    T128 = 128

    trans_m = trans_ref[...]
    tmax = jnp.max(trans_m, axis=0, keepdims=True)
    texp_s[...] = jnp.exp(trans_m - tmax)
    transt_s[...] = trans_m.T
    tmax_s[...] = jnp.broadcast_to(tmax, (8, T128))

    rowi = jax.lax.broadcasted_iota(jnp.int32, (B, T128), 1)
    em0 = emt_ref[0]
    lab0 = lbc_ref[0].astype(jnp.int32)
    startv = start_ref[...]
    score_s[...] = startv + em0
    vscore_s[...] = startv + em0
    numvec_s[...] = jnp.where(rowi == lab0, em0 + startv, 0.0)
    lt_s[...] = lab0

    def body(s, _):
        em_i = emt_ref[s]
        mb = mbc_ref[s].astype(jnp.int32) > 0
        lab = lbc_ref[s].astype(jnp.int32)
        labp = lbc_ref[s - 1].astype(jnp.int32)
        score = score_s[...]
        smax = jnp.max(score, axis=1, keepdims=True)
        p = jnp.exp(score - smax)
        m = jnp.dot(p, texp_s[...], preferred_element_type=jnp.float32)
        lse = smax + tmax_s[0:1, :] + jnp.log(m)
        score_s[...] = jnp.where(mb, lse + em_i, score)
        vscore = vscore_s[...]
        hist_s[pl.ds(s - 1, 1)] = vscore.reshape(1, B, T128)
        vmx_parts = []
        for jt in range(9):
            tt = transt_s[jt * 8:(jt + 1) * 8, :]
            cand_t = vscore[:, None, :] + tt[None, :, :]
            vmx_parts.append(jnp.max(cand_t, axis=2))
        vmx = jnp.concatenate(vmx_parts, axis=1)
        vmx = jnp.pad(vmx, ((0, 0), (0, T128 - 72)),
                      constant_values=_NEG)
        vscore_s[...] = jnp.where(mb, vmx + em_i, vscore)
        ohp = (rowi == labp).astype(jnp.float32)
        r = jnp.dot(ohp, trans_m, preferred_element_type=jnp.float32)
        numvec_s[...] += jnp.where((rowi == lab) & mb, em_i + r, 0.0)
        lt_s[...] = jnp.where(mb, lab, lt_s[...])
        return 0

    jax.lax.fori_loop(1, S, body, 0)

    endv = end_ref[...]
    x = score_s[...] + endv
    dmax = jnp.max(x, axis=1, keepdims=True)
    denom = dmax + jnp.log(jnp.sum(jnp.exp(x - dmax), axis=1, keepdims=True))
    numvec = numvec_s[...] + jnp.where(rowi == lt_s[...],
                                       jnp.broadcast_to(endv, (B, T128)), 0.0)
    num = jnp.sum(numvec, axis=1, keepdims=True)
    llh_ref[...] = num - denom
    vx = vscore_s[...] + endv
    best = jnp.argmax(vx, axis=1, keepdims=True)
    best_bc = jnp.broadcast_to(best, (B, T128))
    mlast = mbc_ref[S - 1].astype(jnp.int32) > 0
    tags_ref[pl.ds(S - 1, 1)] = jnp.where(mlast, best_bc, 0).astype(
        jnp.int8).reshape(1, B, T128)

    def back(r, carry):
        t = S - 2 - r
        vt = hist_s[pl.ds(t, 1)].reshape(B, T128)
        ohc = (rowi == carry).astype(jnp.float32)
        tcol = jnp.dot(ohc, transt_s[...],
                       preferred_element_type=jnp.float32,
                       precision=jax.lax.Precision.HIGHEST)
        prev = jnp.argmax(vt + tcol, axis=1, keepdims=True)
        prev_bc = jnp.broadcast_to(prev, (B, T128))
        msel = mbc_ref[t + 1].astype(jnp.int32) > 0
        prev_bc = jnp.where(msel, prev_bc, carry)
        mt = mbc_ref[t].astype(jnp.int32) > 0
        tags_ref[pl.ds(t, 1)] = jnp.where(mt, prev_bc, 0).astype(
            jnp.int8).reshape(1, B, T128)
        return prev_bc

    jax.lax.fori_loop(0, S - 1, back, best_bc)


def kernel(hidden, labels, mask, W, b, start_trans, end_trans, trans):
    B, S, H = hidden.shape
    T = W.shape[1]
    PT = 128 - T

    w_pad = jnp.pad(W, ((0, 0), (0, PT)))
    b_pad = jnp.pad(b, (0, PT), constant_values=_NEG).reshape(1, 128)

    sb = 32
    em, emt = pl.pallas_call(
        _em_kernel,
        out_shape=(
            jax.ShapeDtypeStruct((B, S, T), jnp.float32),
            jax.ShapeDtypeStruct((S, B, 128), jnp.float32),
        ),
        grid=(S // sb,),
        in_specs=[
            pl.BlockSpec((B, sb, H), lambda j: (0, j, 0)),
            pl.BlockSpec((H, 128), lambda j: (0, 0)),
            pl.BlockSpec((1, 128), lambda j: (0, 0)),
        ],
        out_specs=(
            pl.BlockSpec((B, sb, T), lambda j: (0, j, 0)),
            pl.BlockSpec((sb, B, 128), lambda j: (j, 0, 0)),
        ),
        compiler_params=pltpu.CompilerParams(
            dimension_semantics=("arbitrary",),
        ),
        name="em_matmul",
    )(hidden, w_pad, b_pad)

    start_pad = jnp.pad(start_trans, (0, PT), constant_values=_NEG).reshape(1, 128)
    end_pad = jnp.pad(end_trans, (0, PT), constant_values=_NEG).reshape(1, 128)
    trans_pad = jnp.pad(trans, ((0, PT), (0, PT)), constant_values=_NEG)
    mbc = jnp.broadcast_to(
        mask.astype(jnp.int8).T[:, :, None], (S, B, 128))
    lbc = jnp.broadcast_to(
        labels.astype(jnp.int8).T[:, :, None], (S, B, 128))

    llh, tags_raw = pl.pallas_call(
        _crf_kernel,
        out_shape=(
            jax.ShapeDtypeStruct((B, 1), jnp.float32),
            jax.ShapeDtypeStruct((S, B, 128), jnp.int8),
        ),
        grid=(1,),
        in_specs=[
            pl.BlockSpec((S, B, 128), lambda i: (0, 0, 0)),
            pl.BlockSpec((S, B, 128), lambda i: (0, 0, 0)),
            pl.BlockSpec((S, B, 128), lambda i: (0, 0, 0)),
            pl.BlockSpec((1, 128), lambda i: (0, 0)),
            pl.BlockSpec((1, 128), lambda i: (0, 0)),
            pl.BlockSpec((128, 128), lambda i: (0, 0)),
        ],
        out_specs=(
            pl.BlockSpec((B, 1), lambda i: (0, 0)),
            pl.BlockSpec((S, B, 128), lambda i: (0, 0, 0)),
        ),
        scratch_shapes=[
            pltpu.VMEM((B, 128), jnp.float32),
            pltpu.VMEM((B, 128), jnp.float32),
            pltpu.VMEM((B, 128), jnp.float32),
            pltpu.VMEM((B, 128), jnp.int32),
            pltpu.VMEM((S - 1, B, 128), jnp.float32),
            pltpu.VMEM((128, 128), jnp.float32),
            pltpu.VMEM((128, 128), jnp.float32),
            pltpu.VMEM((8, 128), jnp.float32),
        ],
        compiler_params=pltpu.CompilerParams(
            dimension_semantics=("arbitrary",),
            vmem_limit_bytes=56 * 1024 * 1024,
        ),
        name="crf_fused",
    )(emt, mbc, lbc, start_pad, end_pad, trans_pad)

    loss = -jnp.sum(llh) / B
    tags = tags_raw[:, :, 0].astype(jnp.int32).T
    return loss, tags, em
